# baseline (speedup 1.0000x reference)
import jax
import jax.numpy as jnp
from jax.experimental import pallas as pl
from jax.experimental.pallas import tpu as pltpu

DZ = 32

EPS1 = -0.02830461598932743


def _body(x_ref, act_ref,
          we1_ref, be1_ref, we2_ref, be2_ref, we3_ref, be3_ref,
          wt1_ref, bt1_ref, wv_ref, bv_ref, wr_ref, br_ref, wo_ref, bo_ref,
          bmat_ref, wd1_ref, bd1_ref, wd2_ref, bd2_ref, wd3_ref, bd3_ref,
          out_ref,
          gwe1, gwe2, gwe3, gwt1, gwv, gwr, gwo, gbmat, gwd1, gwd2, gwd3,
          sbe1, sbe2, sbe3, sbt1, sbv, sbr, sbo, sbd1, sbd2, sbd3,
          swe1, swe2, swe3, swt1, swvro, sbmat, swd1, swd2, swd3,
          sem):
    f32 = jnp.float32
    bf16 = jnp.bfloat16

    @pl.when(pl.program_id(0) == 0)
    def _load_weights():
        pairs = [
            (we1_ref, gwe1), (we2_ref, gwe2), (we3_ref, gwe3),
            (wt1_ref, gwt1), (wv_ref, gwv), (wr_ref, gwr), (wo_ref, gwo),
            (bmat_ref, gbmat), (wd1_ref, gwd1), (wd2_ref, gwd2),
            (wd3_ref, gwd3),
            (be1_ref, sbe1), (be2_ref, sbe2), (be3_ref, sbe3),
            (bt1_ref, sbt1), (bv_ref, sbv), (br_ref, sbr), (bo_ref, sbo),
            (bd1_ref, sbd1), (bd2_ref, sbd2), (bd3_ref, sbd3),
        ]
        copies = [pltpu.make_async_copy(src, dst, sem) for src, dst in pairs]
        for c in copies:
            c.start()
        for c in copies:
            c.wait()
        swe1[...] = gwe1[...].astype(bf16)
        swe2[...] = gwe2[...].astype(bf16)
        swe3[...] = gwe3[...].astype(bf16)
        swt1[...] = gwt1[...].astype(bf16)
        swvro[:, :DZ] = gwv[...].astype(bf16)
        swvro[:, DZ:2 * DZ] = gwr[...].astype(bf16)
        swvro[:, 2 * DZ:] = gwo[...].astype(bf16)
        sbmat[...] = gbmat[...].astype(bf16)
        swd1[...] = gwd1[...].astype(bf16)
        swd2[...] = gwd2[...].astype(bf16)
        swd3[...] = gwd3[...].astype(bf16)

    def mm(a, w_ref):
        return jax.lax.dot_general(
            a.astype(bf16), w_ref[...],
            (((1,), (0,)), ((), ())),
            preferred_element_type=f32)

    h1 = jnp.maximum(mm(x_ref[...], swe1) + sbe1[...], 0.0)
    h2 = jnp.maximum(mm(h1, swe2) + sbe2[...], 0.0)
    ml = mm(h2, swe3) + sbe3[...]
    mean = ml[:, :DZ]
    logvar = ml[:, DZ:]
    z = mean + EPS1 * jnp.exp(0.5 * logvar)

    h = jnp.maximum(mm(z, swt1) + sbt1[...], 0.0)
    vro = mm(h, swvro)
    v = vro[:, :DZ] + sbv[...]
    r = vro[:, DZ:2 * DZ] + sbr[...]
    o = vro[:, 2 * DZ:] + sbo[...]
    s = jnp.sum(r * z, axis=1, keepdims=True)
    znp = z + v * s + mm(act_ref[...], sbmat) + o

    d1 = jnp.maximum(mm(znp, swd1) + sbd1[...], 0.0)
    d2 = jnp.maximum(mm(d1, swd2) + sbd2[...], 0.0)
    out_ref[...] = jax.nn.sigmoid(mm(d2, swd3) + sbd3[...])


def kernel(x, action, x_next, We1, be1, We2, be2, We3, be3, Wd1, bd1, Wd2,
           bd2, Wd3, bd3, Wt1, bt1, Wv, bv, Wr, br, Wo, bo, Bmat, codebook):
    B, DIN = x.shape
    H = We1.shape[1]
    HT = Wt1.shape[1]
    DU = action.shape[1]
    BM = 1024

    f32 = jnp.float32
    bf16 = jnp.bfloat16
    row = lambda i: (i, 0)
    hbm = pl.BlockSpec(memory_space=pl.ANY)

    grid = B // BM
    out = pl.pallas_call(
        _body,
        grid=(grid,),
        in_specs=[
            pl.BlockSpec((BM, DIN), row),
            pl.BlockSpec((BM, DU), row),
        ] + [hbm] * 21,
        out_specs=pl.BlockSpec((BM, DIN), row),
        out_shape=jax.ShapeDtypeStruct((B, DIN), jnp.float32),
        scratch_shapes=[
            pltpu.VMEM((DIN, H), f32),
            pltpu.VMEM((H, H), f32),
            pltpu.VMEM((H, 2 * DZ), f32),
            pltpu.VMEM((DZ, HT), f32),
            pltpu.VMEM((HT, DZ), f32),
            pltpu.VMEM((HT, DZ), f32),
            pltpu.VMEM((HT, DZ), f32),
            pltpu.VMEM((DU, DZ), f32),
            pltpu.VMEM((DZ, H), f32),
            pltpu.VMEM((H, H), f32),
            pltpu.VMEM((H, DIN), f32),
            pltpu.VMEM((H,), f32),
            pltpu.VMEM((H,), f32),
            pltpu.VMEM((2 * DZ,), f32),
            pltpu.VMEM((HT,), f32),
            pltpu.VMEM((DZ,), f32),
            pltpu.VMEM((DZ,), f32),
            pltpu.VMEM((DZ,), f32),
            pltpu.VMEM((H,), f32),
            pltpu.VMEM((H,), f32),
            pltpu.VMEM((DIN,), f32),
            pltpu.VMEM((DIN, H), bf16),
            pltpu.VMEM((H, H), bf16),
            pltpu.VMEM((H, 2 * DZ), bf16),
            pltpu.VMEM((DZ, HT), bf16),
            pltpu.VMEM((HT, 3 * DZ), bf16),
            pltpu.VMEM((DU, DZ), bf16),
            pltpu.VMEM((DZ, H), bf16),
            pltpu.VMEM((H, H), bf16),
            pltpu.VMEM((H, DIN), bf16),
            pltpu.SemaphoreType.DMA,
        ],
        compiler_params=pltpu.CompilerParams(
            dimension_semantics=("arbitrary",),
        ),
    )(
        x, action,
        We1, be1, We2, be2, We3, be3,
        Wt1, bt1, Wv, bv, Wr, br, Wo, bo, Bmat,
        Wd1, bd1, Wd2, bd2, Wd3, bd3,
    )
    return out
